# baseline (speedup 1.0000x reference)
import jax
import jax.numpy as jnp
from jax import lax
from jax.experimental import pallas as pl
from jax.experimental.pallas import tpu as pltpu

_T = 8192
_NF = 32
_B = 8
_BQ = 128
_CK = 1024
_NBLK = _T // _BQ
_SBQ = 512
_NSBLK = _T // _SBQ
_NEG = -1e30


def _bn_full(x, g, b, eps=1e-5):
    m = jnp.mean(x, axis=0, keepdims=True)
    ms = jnp.mean(x * x, axis=0, keepdims=True)
    scale = lax.rsqrt(ms - m * m + eps) * g
    return x * scale + (b - m * scale)


def _layer_kernel(c0_ref, c1_ref, xe_ref, xd_ref, segc_ref,
                  wp1_ref, wq_ref, wk_ref, wv_ref, wtrans_ref, g1_ref, b1_ref,
                  wq1_ref, wk1_ref, wv1_ref, wkv_ref, gkv_ref, bkv_ref,
                  wt1_ref, g2_ref, b2_ref, wr1_ref, wr2_ref,
                  gr1_ref, br1_ref, gr2_ref, br2_ref, gr3_ref, br3_ref,
                  o_ref, lg_s, q_s, qb_s, kb_s, ve_s):
    f32 = jnp.float32
    bf16 = jnp.bfloat16
    xe = xe_ref[:]
    xd = xd_ref[:]

    dec = xd @ wp1_ref[:]
    dq = dec @ wq_ref[:]
    dk = xe @ wk_ref[:]
    dv = xe @ wv_ref[:]

    nt = (((1,), (1,)), ((), ()))
    nn = (((1,), (0,)), ((), ()))
    dv16e = jnp.concatenate(
        [dv.astype(bf16), jnp.ones((_T, 1), bf16)], axis=1)

    q_s[:] = dq

    def xattn(i, _):
        q = q_s[pl.ds(i * _BQ, _BQ), :]
        logits = lax.dot_general(q, dk, nt, preferred_element_type=f32)
        m = jnp.max(logits, axis=-1, keepdims=True)
        e16 = jnp.exp((logits - m).astype(bf16))
        acc = lax.dot_general(e16, dv16e, nn, preferred_element_type=f32)
        o_ref[pl.ds(i * _BQ, _BQ), :] = acc[:, :_NF] / acc[:, _NF:_NF + 1]
        return 0

    lax.fori_loop(0, _NBLK, xattn, 0, unroll=2)
    dec = dec + _bn_full(o_ref[:] @ wtrans_ref[:], g1_ref[:], b1_ref[:])

    segc = segc_ref[:]
    iota_tb = lax.broadcasted_iota(jnp.int32, (_T, _B), 1)
    onehot = segc == iota_tb
    starts = jnp.sum((segc < iota_tb).astype(jnp.int32), axis=0, keepdims=True)
    seg_start_c = jnp.sum(jnp.where(onehot, starts, 0), axis=1, keepdims=True)
    rows = lax.broadcasted_iota(jnp.int32, (_T, 1), 0)
    sel_c = ((rows - seg_start_c) % 2) == 0

    q1 = dec @ wq1_ref[:]
    z = q1 @ wkv_ref[:]
    maskf = sel_c.astype(f32)
    n_sel = jnp.sum(maskf)
    zm = z * maskf
    m_kv = jnp.sum(zm, axis=0, keepdims=True) / n_sel
    ms_kv = jnp.sum(zm * z, axis=0, keepdims=True) / n_sel
    scale_kv = lax.rsqrt(ms_kv - m_kv * m_kv + 1e-5) * gkv_ref[:]
    kv = z * scale_kv + (bkv_ref[:] - m_kv * scale_kv)

    qb_s[:, :_NF] = q1
    qb_s[:, _NF:] = jnp.where(onehot, 1.0, 0.0)
    kb_s[:, :_NF] = kv @ wk1_ref[:]
    kb_s[:, _NF:] = jnp.where(onehot & sel_c, 0.0, _NEG)
    ve_s[:, :_NF] = (kv @ wv1_ref[:]).astype(bf16)
    ve_s[:, _NF:] = jnp.ones((_T, 1), bf16)

    def sattn(i, _):
        qb = qb_s[pl.ds(i * _SBQ, _SBQ), :]

        def chunk(c, carry):
            m0, acc0 = carry
            kb = kb_s[pl.ds(c * _CK, _CK), :]
            ve = ve_s[pl.ds(c * _CK, _CK), :]
            logits = lax.dot_general(qb, kb, nt, preferred_element_type=f32)
            m1 = jnp.maximum(m0, jnp.max(logits, axis=-1, keepdims=True))
            alpha = jnp.exp(m0 - m1)
            e16 = jnp.exp((logits - m1).astype(bf16))
            pv = lax.dot_general(e16, ve, nn, preferred_element_type=f32)
            return m1, acc0 * alpha + pv

        m0 = jnp.full((_SBQ, 1), _NEG, f32)
        acc0 = jnp.zeros((_SBQ, _NF + 1), f32)
        _, acc = lax.fori_loop(c0_ref[i], c1_ref[i], chunk, (m0, acc0))
        o_ref[pl.ds(i * _SBQ, _SBQ), :] = acc[:, :_NF] / acc[:, _NF:_NF + 1]
        return 0

    lax.fori_loop(0, _NSBLK, sattn, 0, unroll=False)
    dec = dec + _bn_full(o_ref[:] @ wt1_ref[:], g2_ref[:], b2_ref[:])

    h = jax.nn.relu(_bn_full(dec, gr1_ref[:], br1_ref[:]))
    h = h @ wr1_ref[:]
    h = jax.nn.relu(_bn_full(h, gr2_ref[:], br2_ref[:]))
    h = h @ wr2_ref[:]
    o_ref[:] = jax.nn.relu(_bn_full(dec + h, gr3_ref[:], br3_ref[:]))


@jax.jit
def kernel(x_encoder, x_decoder, enc_seg, dec_seg, W_p1, W_q, W_k, W_v,
           W_trans, g1, b1, W_q1, W_k1, W_v1, W_kv, g_kv, b_kv, W_t1, g2, b2,
           W_r1, W_r2, g_r1, b_r1, g_r2, b_r2, g_r3, b_r3):
    del enc_seg
    f32 = jnp.float32
    seg = dec_seg.astype(jnp.int32)
    segc = seg.reshape(_T, 1)

    seg_ids = jnp.arange(_B, dtype=jnp.int32)
    seg_lo = jnp.searchsorted(seg, seg_ids, side="left").astype(jnp.int32)
    seg_hi = jnp.searchsorted(seg, seg_ids, side="right").astype(jnp.int32)
    blk_first = seg[:: _SBQ]
    blk_last = seg[_SBQ - 1:: _SBQ]
    c0 = seg_lo[blk_first] // _CK
    c1 = (seg_hi[blk_last] + _CK - 1) // _CK

    row = lambda v: v.astype(f32).reshape(1, _NF)
    args = (c0, c1, x_encoder, x_decoder, segc,
            W_p1, W_q, W_k, W_v, W_trans, row(g1), row(b1),
            W_q1, W_k1, W_v1, W_kv, row(g_kv), row(b_kv),
            W_t1, row(g2), row(b2), W_r1, W_r2,
            row(g_r1), row(b_r1), row(g_r2), row(b_r2), row(g_r3), row(b_r3))
    smem = pl.BlockSpec(memory_space=pltpu.SMEM)
    in_specs = [smem, smem] + [pl.BlockSpec() for _ in range(len(args) - 2)]
    return pl.pallas_call(
        _layer_kernel,
        out_shape=jax.ShapeDtypeStruct((_T, _NF), f32),
        in_specs=in_specs,
        scratch_shapes=[pltpu.VMEM((2 * _BQ, _T), f32),
                        pltpu.VMEM((_T, _NF), f32),
                        pltpu.VMEM((_T, _NF + _B), f32),
                        pltpu.VMEM((_T, _NF + _B), f32),
                        pltpu.VMEM((_T, _NF + 1), jnp.bfloat16)],
        compiler_params=pltpu.CompilerParams(
            vmem_limit_bytes=63 * 1024 * 1024),
    )(*args)


# speedup vs baseline: 1.0852x; 1.0852x over previous
import jax
import jax.numpy as jnp
from jax import lax
from jax.experimental import pallas as pl
from jax.experimental.pallas import tpu as pltpu

_T = 8192
_NF = 32
_B = 8
_BQ = 256
_CK = 1024
_NBLK = _T // _BQ
_SBQ = 512
_NSBLK = _T // _SBQ
_NEG = -1e30


def _bn_full(x, g, b, eps=1e-5):
    m = jnp.mean(x, axis=0, keepdims=True)
    ms = jnp.mean(x * x, axis=0, keepdims=True)
    scale = lax.rsqrt(ms - m * m + eps) * g
    return x * scale + (b - m * scale)


def _layer_kernel(c0_ref, c1_ref, xe_ref, xd_ref, segc_ref,
                  wp1_ref, wq_ref, wk_ref, wv_ref, wtrans_ref, g1_ref, b1_ref,
                  wq1_ref, wk1_ref, wv1_ref, wkv_ref, gkv_ref, bkv_ref,
                  wt1_ref, g2_ref, b2_ref, wr1_ref, wr2_ref,
                  gr1_ref, br1_ref, gr2_ref, br2_ref, gr3_ref, br3_ref,
                  o_ref, lg_s, q_s, qb_s, kb_s, ve_s):
    f32 = jnp.float32
    bf16 = jnp.bfloat16
    xe = xe_ref[:]
    xd = xd_ref[:]

    dec = xd @ wp1_ref[:]
    dq = dec @ wq_ref[:]
    dk = xe @ wk_ref[:]
    dv = xe @ wv_ref[:]

    nt = (((1,), (1,)), ((), ()))
    nn = (((1,), (0,)), ((), ()))
    dv16e = jnp.concatenate(
        [dv.astype(bf16), jnp.ones((_T, 1), bf16)], axis=1)

    q_s[:] = dq

    def xattn(i, _):
        q = q_s[pl.ds(i * _BQ, _BQ), :]
        logits = lax.dot_general(q, dk, nt, preferred_element_type=f32)
        m = jnp.max(logits, axis=-1, keepdims=True)
        e16 = jnp.exp((logits - m).astype(bf16))
        acc = lax.dot_general(e16, dv16e, nn, preferred_element_type=f32)
        o_ref[pl.ds(i * _BQ, _BQ), :] = acc[:, :_NF] / acc[:, _NF:_NF + 1]
        return 0

    lax.fori_loop(0, _NBLK, xattn, 0, unroll=2)
    dec = dec + _bn_full(o_ref[:] @ wtrans_ref[:], g1_ref[:], b1_ref[:])

    segc = segc_ref[:]
    iota_tb = lax.broadcasted_iota(jnp.int32, (_T, _B), 1)
    onehot = segc == iota_tb
    starts = jnp.sum((segc < iota_tb).astype(jnp.int32), axis=0, keepdims=True)
    seg_start_c = jnp.sum(jnp.where(onehot, starts, 0), axis=1, keepdims=True)
    rows = lax.broadcasted_iota(jnp.int32, (_T, 1), 0)
    sel_c = ((rows - seg_start_c) % 2) == 0

    q1 = dec @ wq1_ref[:]
    z = q1 @ wkv_ref[:]
    maskf = sel_c.astype(f32)
    n_sel = jnp.sum(maskf)
    zm = z * maskf
    m_kv = jnp.sum(zm, axis=0, keepdims=True) / n_sel
    ms_kv = jnp.sum(zm * z, axis=0, keepdims=True) / n_sel
    scale_kv = lax.rsqrt(ms_kv - m_kv * m_kv + 1e-5) * gkv_ref[:]
    kv = z * scale_kv + (bkv_ref[:] - m_kv * scale_kv)

    qb_s[:, :_NF] = q1
    qb_s[:, _NF:] = jnp.where(onehot, 1.0, 0.0)
    kb_s[:, :_NF] = kv @ wk1_ref[:]
    kb_s[:, _NF:] = jnp.where(onehot & sel_c, 0.0, _NEG)
    ve_s[:, :_NF] = (kv @ wv1_ref[:]).astype(bf16)
    ve_s[:, _NF:] = jnp.ones((_T, 1), bf16)

    def sattn(i, _):
        qb = qb_s[pl.ds(i * _SBQ, _SBQ), :]

        def chunk(c, carry):
            m0, acc0 = carry
            kb = kb_s[pl.ds(c * _CK, _CK), :]
            ve = ve_s[pl.ds(c * _CK, _CK), :]
            logits = lax.dot_general(qb, kb, nt, preferred_element_type=f32)
            m1 = jnp.maximum(m0, jnp.max(logits, axis=-1, keepdims=True))
            alpha = jnp.exp(m0 - m1)
            e16 = jnp.exp((logits - m1).astype(bf16))
            pv = lax.dot_general(e16, ve, nn, preferred_element_type=f32)
            return m1, acc0 * alpha + pv

        m0 = jnp.full((_SBQ, 1), _NEG, f32)
        acc0 = jnp.zeros((_SBQ, _NF + 1), f32)
        _, acc = lax.fori_loop(c0_ref[i], c1_ref[i], chunk, (m0, acc0))
        o_ref[pl.ds(i * _SBQ, _SBQ), :] = acc[:, :_NF] / acc[:, _NF:_NF + 1]
        return 0

    lax.fori_loop(0, _NSBLK, sattn, 0, unroll=2)
    dec = dec + _bn_full(o_ref[:] @ wt1_ref[:], g2_ref[:], b2_ref[:])

    h = jax.nn.relu(_bn_full(dec, gr1_ref[:], br1_ref[:]))
    h = h @ wr1_ref[:]
    h = jax.nn.relu(_bn_full(h, gr2_ref[:], br2_ref[:]))
    h = h @ wr2_ref[:]
    o_ref[:] = jax.nn.relu(_bn_full(dec + h, gr3_ref[:], br3_ref[:]))


@jax.jit
def kernel(x_encoder, x_decoder, enc_seg, dec_seg, W_p1, W_q, W_k, W_v,
           W_trans, g1, b1, W_q1, W_k1, W_v1, W_kv, g_kv, b_kv, W_t1, g2, b2,
           W_r1, W_r2, g_r1, b_r1, g_r2, b_r2, g_r3, b_r3):
    del enc_seg
    f32 = jnp.float32
    seg = dec_seg.astype(jnp.int32)
    segc = seg.reshape(_T, 1)

    seg_ids = jnp.arange(_B, dtype=jnp.int32)
    seg_lo = jnp.searchsorted(seg, seg_ids, side="left").astype(jnp.int32)
    seg_hi = jnp.searchsorted(seg, seg_ids, side="right").astype(jnp.int32)
    blk_first = seg[:: _SBQ]
    blk_last = seg[_SBQ - 1:: _SBQ]
    c0 = seg_lo[blk_first] // _CK
    c1 = (seg_hi[blk_last] + _CK - 1) // _CK

    row = lambda v: v.astype(f32).reshape(1, _NF)
    args = (c0, c1, x_encoder, x_decoder, segc,
            W_p1, W_q, W_k, W_v, W_trans, row(g1), row(b1),
            W_q1, W_k1, W_v1, W_kv, row(g_kv), row(b_kv),
            W_t1, row(g2), row(b2), W_r1, W_r2,
            row(g_r1), row(b_r1), row(g_r2), row(b_r2), row(g_r3), row(b_r3))
    smem = pl.BlockSpec(memory_space=pltpu.SMEM)
    in_specs = [smem, smem] + [pl.BlockSpec() for _ in range(len(args) - 2)]
    return pl.pallas_call(
        _layer_kernel,
        out_shape=jax.ShapeDtypeStruct((_T, _NF), f32),
        in_specs=in_specs,
        scratch_shapes=[pltpu.VMEM((2 * _BQ, _T), f32),
                        pltpu.VMEM((_T, _NF), f32),
                        pltpu.VMEM((_T, _NF + _B), f32),
                        pltpu.VMEM((_T, _NF + _B), f32),
                        pltpu.VMEM((_T, _NF + 1), jnp.bfloat16)],
        compiler_params=pltpu.CompilerParams(
            vmem_limit_bytes=63 * 1024 * 1024),
    )(*args)
